# baseline (speedup 1.0000x reference)
import functools

import jax
import jax.numpy as jnp
from jax import lax
from jax.experimental import pallas as pl
from jax.experimental.pallas import tpu as pltpu
from jax.experimental.pallas import tpu_sc as plsc

_NC = 2
_NS = 16
_NW = _NC * _NS

_D = 64
_K = 10
_KP = 16

_B = 4096
_T = 200
_N = _B * _T

_GRP = 128
_NG = _N // _GRP
_GPW = _NG // _NW
_L = 16


def _proj_body(tab_ref, w_ref, b_ref, out_ref):
    out_ref[...] = (
        jnp.dot(tab_ref[...], w_ref[...], preferred_element_type=jnp.float32)
        + b_ref[...]
    )


def _project_table(embed_table, w_pad, b_pad):
    V, D = embed_table.shape
    blk = 4000
    return pl.pallas_call(
        _proj_body,
        grid=(V // blk,),
        in_specs=[
            pl.BlockSpec((blk, D), lambda i: (i, 0)),
            pl.BlockSpec((D, _KP), lambda i: (0, 0)),
            pl.BlockSpec((1, _KP), lambda i: (0, 0)),
        ],
        out_specs=pl.BlockSpec((blk, _KP), lambda i: (i, 0)),
        out_shape=jax.ShapeDtypeStruct((V, _KP), jnp.float32),
    )(embed_table, w_pad, b_pad)


def _sc_gather_transpose(embed_table, proj_table, ids2d):
    mesh = plsc.VectorSubcoreMesh(
        core_axis_name="c", subcore_axis_name="s",
        num_cores=_NC, num_subcores=_NS,
    )

    @functools.partial(
        pl.kernel,
        mesh=mesh,
        out_type=(
            jax.ShapeDtypeStruct((_T, 8, _B // 128, 8, 128), jnp.float32),
            jax.ShapeDtypeStruct((_K, _T // 8, _B // 128, 8, 128),
                                 jnp.float32),
        ),
        scratch_types=[
            pltpu.VMEM((_GPW, _GRP), jnp.int32),
            pltpu.VMEM((_GRP, _D), jnp.float32),
            pltpu.VMEM((_GRP, _D), jnp.float32),
            pltpu.VMEM((_GRP, _KP), jnp.float32),
            pltpu.VMEM((_GRP, _KP), jnp.float32),
            pltpu.VMEM((8, 8, 128), jnp.float32),
            pltpu.VMEM((8, 8, 128), jnp.float32),
            pltpu.VMEM((_K, 128), jnp.float32),
            pltpu.VMEM((_K, 128), jnp.float32),
            pltpu.SemaphoreType.DMA,
            pltpu.SemaphoreType.DMA,
            pltpu.SemaphoreType.DMA,
            pltpu.SemaphoreType.DMA,
        ],
        compiler_params=pltpu.CompilerParams(
            use_tc_tiling_on_sc=False, needs_layout_passes=False),
    )
    def gather_kernel(tab_hbm, ptab_hbm, ids_hbm, emb_out, log_out,
                      idx_v, eb0, eb1, pb0, pb1, tl0, tl1, lt0, lt1,
                      gs0, gs1, ws0, ws1):
        wid = lax.axis_index("s") * _NC + lax.axis_index("c")
        g_base = pl.multiple_of(wid * _GPW, 8)

        pltpu.sync_copy(ids_hbm.at[pl.ds(g_base, _GPW)], idx_v)

        rows = [lax.broadcasted_iota(jnp.int32, (_L,), 0) + bc * _L
                for bc in range(_GRP // _L)]

        def fire(g, eb, pb, gs):
            pltpu.async_copy(tab_hbm.at[idx_v.at[g]], eb, gs)
            pltpu.async_copy(ptab_hbm.at[idx_v.at[g]], pb, gs)

        def wait_gather(eb, pb, gs):
            pltpu.make_async_copy(tab_hbm.at[pl.ds(0, _GRP)], eb, gs).wait()
            pltpu.make_async_copy(ptab_hbm.at[pl.ds(0, _GRP)], pb, gs).wait()

        def emb_dst(g):
            n0 = (g_base + g) * _GRP
            t = n0 // _B
            b32 = (n0 - t * _B) // 128
            return emb_out.at[t, :, b32]

        def log_dst(g):
            n0 = (g_base + g) * _GRP
            t = n0 // _B
            b32 = (n0 - t * _B) // 128
            return log_out.at[pl.ds(0, _K), t // 8, b32, t % 8]

        def wait_write(g, tl, lt, ws):
            pltpu.make_async_copy(tl, emb_dst(g), ws).wait()
            pltpu.make_async_copy(lt, log_dst(g), ws).wait()

        def process(g, eb, pb, tl, lt, ws):
            for d8 in range(8):
                for dsub in range(8):
                    col = jnp.full((_L,), d8 * 8 + dsub, jnp.int32)
                    for bc in range(_GRP // _L):
                        v = plsc.load_gather(eb, [rows[bc], col])
                        tl[d8, dsub, pl.ds(bc * _L, _L)] = v
            for k in range(_K):
                colk = jnp.full((_L,), k, jnp.int32)
                for bc in range(_GRP // _L):
                    v = plsc.load_gather(pb, [rows[bc], colk])
                    lt[k, pl.ds(bc * _L, _L)] = v
            pltpu.async_copy(tl, emb_dst(g), ws)
            pltpu.async_copy(lt, log_dst(g), ws)

        fire(0, eb0, pb0, gs0)

        def body(i, _):
            g0 = i * 2
            g1 = g0 + 1
            fire(g1, eb1, pb1, gs1)
            wait_gather(eb0, pb0, gs0)

            @pl.when(i > 0)
            def _():
                wait_write(g0 - 2, tl0, lt0, ws0)
            process(g0, eb0, pb0, tl0, lt0, ws0)

            @pl.when(i < _GPW // 2 - 1)
            def _():
                fire(g0 + 2, eb0, pb0, gs0)
            wait_gather(eb1, pb1, gs1)

            @pl.when(i > 0)
            def _():
                wait_write(g1 - 2, tl1, lt1, ws1)
            process(g1, eb1, pb1, tl1, lt1, ws1)
            return ()

        lax.fori_loop(0, _GPW // 2, body, ())
        wait_write(_GPW - 2, tl0, lt0, ws0)
        wait_write(_GPW - 1, tl1, lt1, ws1)

    return gather_kernel(embed_table, proj_table, ids2d)


def kernel(text_ids, embed_table, dur_w, dur_b):
    ids2d = jnp.swapaxes(text_ids, 0, 1).reshape(_NG, _GRP)
    w_pad = jnp.pad(dur_w, ((0, 0), (0, _KP - _K)))
    b_pad = jnp.pad(dur_b, (0, _KP - _K)).reshape(1, _KP)
    proj_table = _project_table(embed_table, w_pad, b_pad)
    emb5d, log5d = _sc_gather_transpose(embed_table, proj_table, ids2d)
    embeddings = (emb5d.transpose(0, 1, 3, 2, 4)
                  .reshape(_T, _D, _B).transpose(2, 0, 1))
    logits = (log5d.transpose(0, 1, 3, 2, 4)
              .reshape(_K, _T, _B).transpose(2, 1, 0))
    return (embeddings, logits)


# speedup vs baseline: 4.3799x; 4.3799x over previous
import functools

import jax
import jax.numpy as jnp
from jax import lax
from jax.experimental import pallas as pl
from jax.experimental.pallas import tpu as pltpu
from jax.experimental.pallas import tpu_sc as plsc

_NC = 2
_NS = 16
_NW = _NC * _NS

_D = 64
_K = 10
_KP = 16

_B = 4096
_T = 200
_N = _B * _T

_GRP = 128
_GPC = 8
_CHUNK = _GRP * _GPC
_HALF = 512
_GBLK = 2 * _HALF

_NSLICE = 5
_TS = _T // _NSLICE
_TG = 8


def _sc_gather_slice(embed_table, ids2d, t0):
    n_groups = _TS * (_B // _GRP)
    per_w = n_groups // _NW
    n_chunks = per_w // _GPC
    mesh = plsc.VectorSubcoreMesh(
        core_axis_name="c", subcore_axis_name="s",
        num_cores=_NC, num_subcores=_NS,
    )

    @functools.partial(
        pl.kernel,
        mesh=mesh,
        out_type=jax.ShapeDtypeStruct((_TS, _B // _GBLK, _HALF, 2 * _D),
                                      jnp.float32),
        scratch_types=[
            pltpu.VMEM((_GPC, _GRP), jnp.int32),
            pltpu.VMEM((_CHUNK, _D), jnp.float32),
            pltpu.SemaphoreType.DMA,
        ],
        compiler_params=pltpu.CompilerParams(use_tc_tiling_on_sc=False),
    )
    def gather_kernel(tab_hbm, ids_hbm, out_hbm, idx_v, buf_v, sem):
        wid = lax.axis_index("s") * _NC + lax.axis_index("c")
        g_base = wid * per_w

        def body(c, _):
            g0 = pl.multiple_of(g_base + c * _GPC, _GPC)
            pltpu.sync_copy(
                ids_hbm.at[pl.ds(t0 * (_B // _GRP) + g0, _GPC)], idx_v)
            copies = [
                pltpu.async_copy(
                    tab_hbm.at[idx_v.at[j]],
                    buf_v.at[pl.ds(j * _GRP, _GRP)], sem)
                for j in range(_GPC)
            ]
            for cp in copies:
                cp.wait()
            for j in range(_GPC):
                n0 = (g0 + j) * _GRP
                t = n0 // _B
                b0 = n0 - t * _B
                g = b0 // _GBLK
                half = (b0 // _HALF) % 2
                r0 = pl.multiple_of(b0 % _HALF, _GRP)
                pltpu.sync_copy(
                    buf_v.at[pl.ds(j * _GRP, _GRP)],
                    out_hbm.at[t, g, pl.ds(r0, _GRP),
                               pl.ds(half * _D, _D)])
            return ()

        lax.fori_loop(0, n_chunks, body, ())

    return gather_kernel(embed_table, ids2d)


def _tc_body(x_ref, wt_ref, b_ref, *rest):
    emb_ref, log_ref = rest[-2], rest[-1]
    for t in range(_TG):
        for g in range(_B // _GBLK):
            x = x_ref[t, g]
            xt = jnp.transpose(x, (1, 0))
            e = xt[:_D, :]
            o = xt[_D:, :]
            emb_ref[t, :, g * _GBLK:g * _GBLK + _HALF] = e
            emb_ref[t, :, g * _GBLK + _HALF:(g + 1) * _GBLK] = o
            wt = wt_ref[...]
            bias = b_ref[...]
            le = jnp.dot(wt, e, preferred_element_type=jnp.float32) + bias
            lo = jnp.dot(wt, o, preferred_element_type=jnp.float32) + bias
            log_ref[:, t, g * _GBLK:g * _GBLK + _HALF] = le[:_K, :]
            log_ref[:, t, g * _GBLK + _HALF:(g + 1) * _GBLK] = lo[:_K, :]


def _tc_finish_slice(staged, wt_pad, b_pad, t0, prev):
    n_gb = _B // _GBLK
    grid = (_TS // _TG,)
    tb0 = t0 // _TG
    in_specs = [
        pl.BlockSpec((_TG, n_gb, _HALF, 2 * _D), lambda i: (i, 0, 0, 0)),
        pl.BlockSpec((_KP, _D), lambda i: (0, 0)),
        pl.BlockSpec((_KP, 1), lambda i: (0, 0)),
    ]
    out_specs = [
        pl.BlockSpec((_TG, _D, _B), lambda i: (tb0 + i, 0, 0)),
        pl.BlockSpec((_K, _TG, _B), lambda i: (0, tb0 + i, 0)),
    ]
    out_shape = [
        jax.ShapeDtypeStruct((_T, _D, _B), jnp.float32),
        jax.ShapeDtypeStruct((_K, _T, _B), jnp.float32),
    ]
    args = [staged, wt_pad, b_pad]
    kwargs = {}
    if prev is not None:
        in_specs += [pl.BlockSpec(memory_space=pl.ANY),
                     pl.BlockSpec(memory_space=pl.ANY)]
        args += [prev[0], prev[1]]
        kwargs["input_output_aliases"] = {3: 0, 4: 1}
    return pl.pallas_call(
        _tc_body,
        grid=grid,
        in_specs=in_specs,
        out_specs=out_specs,
        out_shape=out_shape,
        **kwargs,
    )(*args)


def kernel(text_ids, embed_table, dur_w, dur_b):
    ids2d = jnp.swapaxes(text_ids, 0, 1).reshape(_N // _GRP, _GRP)
    wt_pad = jnp.pad(jnp.transpose(dur_w), ((0, _KP - _K), (0, 0)))
    b_pad = jnp.pad(dur_b, (0, _KP - _K)).reshape(_KP, 1)
    staged = [_sc_gather_slice(embed_table, ids2d, s * _TS)
              for s in range(_NSLICE)]
    prev = None
    for s in range(_NSLICE):
        prev = _tc_finish_slice(staged[s], wt_pad, b_pad, s * _TS, prev)
    emb_t, log_t = prev
    embeddings = jnp.transpose(emb_t, (2, 0, 1))
    logits = jnp.transpose(log_t, (2, 1, 0))
    return (embeddings, logits)


# speedup vs baseline: 4.5981x; 1.0498x over previous
import functools

import jax
import jax.numpy as jnp
from jax import lax
from jax.experimental import pallas as pl
from jax.experimental.pallas import tpu as pltpu
from jax.experimental.pallas import tpu_sc as plsc

_NC = 2
_NS = 16
_NW = _NC * _NS

_D = 64
_K = 10
_KP = 16

_B = 4096
_T = 200
_N = _B * _T

_GRP = 128
_GPC = 8
_CHUNK = _GRP * _GPC
_HALF = 512
_GBLK = 2 * _HALF

_NSLICE = 5
_TS = _T // _NSLICE
_TG = 8


def _sc_gather_slice(embed_table, ids2d, t0):
    n_groups = _TS * (_B // _GRP)
    per_w = n_groups // _NW
    n_chunks = per_w // _GPC
    mesh = plsc.VectorSubcoreMesh(
        core_axis_name="c", subcore_axis_name="s",
        num_cores=_NC, num_subcores=_NS,
    )

    gpc = 4
    n_chunks = per_w // gpc

    @functools.partial(
        pl.kernel,
        mesh=mesh,
        out_type=jax.ShapeDtypeStruct((_TS, _B // _GBLK, _HALF, 2 * _D),
                                      jnp.float32),
        scratch_types=[
            pltpu.VMEM((per_w, _GRP), jnp.int32),
            pltpu.VMEM((gpc * _GRP, _D), jnp.float32),
            pltpu.VMEM((gpc * _GRP, _D), jnp.float32),
            pltpu.SemaphoreType.DMA,
            pltpu.SemaphoreType.DMA,
        ],
        compiler_params=pltpu.CompilerParams(use_tc_tiling_on_sc=False),
    )
    def gather_kernel(tab_hbm, ids_hbm, out_hbm, idx_v, bufa, bufb,
                      sema, semb):
        wid = lax.axis_index("s") * _NC + lax.axis_index("c")
        g_base = pl.multiple_of(wid * per_w, 8)
        pltpu.sync_copy(
            ids_hbm.at[pl.ds(t0 * (_B // _GRP) + g_base, per_w)], idx_v)

        def fire(c, buf, sem):
            for j in range(gpc):
                pltpu.async_copy(tab_hbm.at[idx_v.at[c * gpc + j]],
                                 buf.at[pl.ds(j * _GRP, _GRP)], sem)

        def wait_gather(buf, sem):
            for j in range(gpc):
                pltpu.make_async_copy(tab_hbm.at[pl.ds(0, _GRP)],
                                      buf.at[pl.ds(j * _GRP, _GRP)],
                                      sem).wait()

        def writeout(c, buf):
            for j in range(gpc):
                n0 = (g_base + c * gpc + j) * _GRP
                t = n0 // _B
                b0 = n0 - t * _B
                g = b0 // _GBLK
                half = (b0 // _HALF) % 2
                r0 = pl.multiple_of(b0 % _HALF, _GRP)
                pltpu.sync_copy(
                    buf.at[pl.ds(j * _GRP, _GRP)],
                    out_hbm.at[t, g, pl.ds(r0, _GRP),
                               pl.ds(half * _D, _D)])

        fire(0, bufa, sema)

        def body(i, _):
            c0 = i * 2
            fire(c0 + 1, bufb, semb)
            wait_gather(bufa, sema)
            writeout(c0, bufa)

            @pl.when(i < n_chunks // 2 - 1)
            def _():
                fire(c0 + 2, bufa, sema)
            wait_gather(bufb, semb)
            writeout(c0 + 1, bufb)
            return ()

        lax.fori_loop(0, n_chunks // 2, body, ())

    return gather_kernel(embed_table, ids2d)


def _tc_body(x_ref, wt_ref, b_ref, *rest):
    emb_ref, log_ref = rest[-2], rest[-1]
    for t in range(_TG):
        for g in range(_B // _GBLK):
            x = x_ref[t, g]
            xt = jnp.transpose(x, (1, 0))
            e = xt[:_D, :]
            o = xt[_D:, :]
            emb_ref[t, :, g * _GBLK:g * _GBLK + _HALF] = e
            emb_ref[t, :, g * _GBLK + _HALF:(g + 1) * _GBLK] = o
            wt = wt_ref[...]
            bias = b_ref[...]
            le = jnp.dot(wt, e, preferred_element_type=jnp.float32) + bias
            lo = jnp.dot(wt, o, preferred_element_type=jnp.float32) + bias
            log_ref[:, t, g * _GBLK:g * _GBLK + _HALF] = le[:_K, :]
            log_ref[:, t, g * _GBLK + _HALF:(g + 1) * _GBLK] = lo[:_K, :]


def _tc_finish_slice(staged, wt_pad, b_pad, t0, prev):
    n_gb = _B // _GBLK
    grid = (_TS // _TG,)
    tb0 = t0 // _TG
    in_specs = [
        pl.BlockSpec((_TG, n_gb, _HALF, 2 * _D), lambda i: (i, 0, 0, 0)),
        pl.BlockSpec((_KP, _D), lambda i: (0, 0)),
        pl.BlockSpec((_KP, 1), lambda i: (0, 0)),
    ]
    out_specs = [
        pl.BlockSpec((_TG, _D, _B), lambda i: (tb0 + i, 0, 0)),
        pl.BlockSpec((_K, _TG, _B), lambda i: (0, tb0 + i, 0)),
    ]
    out_shape = [
        jax.ShapeDtypeStruct((_T, _D, _B), jnp.float32),
        jax.ShapeDtypeStruct((_K, _T, _B), jnp.float32),
    ]
    args = [staged, wt_pad, b_pad]
    kwargs = {}
    if prev is not None:
        in_specs += [pl.BlockSpec(memory_space=pl.ANY),
                     pl.BlockSpec(memory_space=pl.ANY)]
        args += [prev[0], prev[1]]
        kwargs["input_output_aliases"] = {3: 0, 4: 1}
    return pl.pallas_call(
        _tc_body,
        grid=grid,
        in_specs=in_specs,
        out_specs=out_specs,
        out_shape=out_shape,
        **kwargs,
    )(*args)


def kernel(text_ids, embed_table, dur_w, dur_b):
    ids2d = jnp.swapaxes(text_ids, 0, 1).reshape(_N // _GRP, _GRP)
    wt_pad = jnp.pad(jnp.transpose(dur_w), ((0, _KP - _K), (0, 0)))
    b_pad = jnp.pad(dur_b, (0, _KP - _K)).reshape(_KP, 1)
    staged = [_sc_gather_slice(embed_table, ids2d, s * _TS)
              for s in range(_NSLICE)]
    prev = None
    for s in range(_NSLICE):
        prev = _tc_finish_slice(staged[s], wt_pad, b_pad, s * _TS, prev)
    emb_t, log_t = prev
    embeddings = jnp.transpose(emb_t, (2, 0, 1))
    logits = jnp.transpose(log_t, (2, 1, 0))
    return (embeddings, logits)
